# baseline (device time: 121432 ns/iter reference)
import jax
import jax.numpy as jnp
from jax import lax
from jax.experimental import pallas as pl
from jax.experimental.pallas import tpu as pltpu

Z = 4
DL = 128
MAXQ = 32


def kernel(x, dest):
    t, d = x.shape
    dr = t // DL
    dest2 = dest.reshape(dr, DL)

    def body(x_ref, d_ref, out_ref, cmat_ref, cnt_s, dest_s, runn_s,
             c_send, c_recv, row_send, in_sem, own_sem, cp_sems):
        mx = lax.axis_index("x")
        my = lax.axis_index("y")
        mz = lax.axis_index("z")

        def row_rdma(src_i, dst_j, dev_z):
            return pltpu.make_async_remote_copy(
                src_ref=x_ref.at[pl.ds(src_i, 1), :],
                dst_ref=out_ref.at[pl.ds(dst_j, 1), :],
                send_sem=row_send,
                recv_sem=in_sem,
                device_id=(mx, my, dev_z),
                device_id_type=pl.DeviceIdType.MESH,
            )

        def own_copy(src_i, dst_j):
            return pltpu.make_async_copy(
                x_ref.at[pl.ds(src_i, 1), :],
                out_ref.at[pl.ds(dst_j, 1), :],
                own_sem,
            )

        barrier = pltpu.get_barrier_semaphore()
        for j in range(1, Z):
            pl.semaphore_signal(
                barrier, inc=1,
                device_id=(mx, my, lax.rem(mz + j, Z)),
                device_id_type=pl.DeviceIdType.MESH,
            )
        pl.semaphore_wait(barrier, Z - 1)

        cp_d = pltpu.make_async_copy(d_ref, dest_s, cp_sems.at[0])
        cp_d.start()

        lane = lax.broadcasted_iota(jnp.int32, (1, DL), 1)
        row = jnp.zeros((1, DL), jnp.int32)
        for r in range(Z):
            c_r = jnp.sum((d_ref[...] == r).astype(jnp.int32))
            row = row + jnp.where(lane == r, c_r, 0)
        cmat_ref[pl.ds(mz, 1), :] = row

        def cnt_rdma(dev_z):
            return pltpu.make_async_remote_copy(
                src_ref=cmat_ref.at[pl.ds(mz, 1), :],
                dst_ref=cmat_ref.at[pl.ds(mz, 1), :],
                send_sem=c_send,
                recv_sem=c_recv,
                device_id=(mx, my, dev_z),
                device_id_type=pl.DeviceIdType.MESH,
            )

        for j in range(1, Z):
            cnt_rdma(lax.rem(mz + j, Z)).start()
        for j in range(Z - 1):
            cnt_rdma(mz).wait_recv()
        for j in range(Z - 1):
            cnt_rdma(mz).wait_send()
        cp_c = pltpu.make_async_copy(cmat_ref, cnt_s, cp_sems.at[1])
        cp_c.start()
        cp_c.wait()
        cp_d.wait()

        for r in range(Z):
            base_r = jnp.int32(0)
            for s in range(Z):
                base_r = base_r + jnp.where(s < mz, cnt_s[s, r], 0)
            runn_s[r] = base_r

        def one(l, carry, rr):
            n_rem, w_rem, n_loc, w_loc = carry
            i = rr * DL + l
            dloc = dest_s[rr, l]
            pos = runn_s[dloc]
            runn_s[dloc] = pos + 1
            is_rem = (dloc != mz).astype(jnp.int32)

            @pl.when(dloc != mz)
            def _():
                row_rdma(i, pos, dloc).start()

            @pl.when(dloc == mz)
            def _():
                own_copy(i, pos).start()

            n_rem = n_rem + is_rem
            n_loc = n_loc + 1 - is_rem
            wr = (n_rem - w_rem >= MAXQ).astype(jnp.int32)
            wl = (n_loc - w_loc >= MAXQ).astype(jnp.int32)

            @pl.when(wr == 1)
            def _():
                row_rdma(0, 0, mz).wait_send()

            @pl.when(wl == 1)
            def _():
                own_copy(0, 0).wait()

            return n_rem, w_rem + wr, n_loc, w_loc + wl

        carry = (jnp.int32(0),) * 4
        for rr in range(dr):
            carry = lax.fori_loop(
                0, DL, lambda l, c, _rr=rr: one(l, c, _rr), carry
            )
        n_rem, w_rem, n_loc, w_loc = carry

        def drain(n, wait_fn):
            def step(_, c):
                wait_fn()
                return c
            lax.fori_loop(0, n, step, 0)

        drain(n_rem - w_rem, lambda: row_rdma(0, 0, mz).wait_send())
        drain(n_loc - w_loc, lambda: own_copy(0, 0).wait())

        n_in = jnp.int32(0)
        for s in range(Z):
            n_in = n_in + jnp.where(s == mz, 0, cnt_s[s, mz])
        drain(n_in, lambda: row_rdma(0, 0, mz).wait_recv())

    return pl.pallas_call(
        body,
        out_shape=jax.ShapeDtypeStruct((t, d), jnp.float32),
        in_specs=[
            pl.BlockSpec(memory_space=pltpu.VMEM),
            pl.BlockSpec(memory_space=pltpu.VMEM),
        ],
        out_specs=pl.BlockSpec(memory_space=pltpu.VMEM),
        scratch_shapes=[
            pltpu.VMEM((Z, DL), jnp.int32),
            pltpu.SMEM((Z, DL), jnp.int32),
            pltpu.SMEM((dr, DL), jnp.int32),
            pltpu.SMEM((Z,), jnp.int32),
            pltpu.SemaphoreType.DMA,
            pltpu.SemaphoreType.DMA,
            pltpu.SemaphoreType.DMA,
            pltpu.SemaphoreType.DMA,
            pltpu.SemaphoreType.DMA,
            pltpu.SemaphoreType.DMA((2,)),
        ],
        compiler_params=pltpu.CompilerParams(
            collective_id=0,
            vmem_limit_bytes=100 * 1024 * 1024,
        ),
    )(x, dest2)


# device time: 119600 ns/iter; 1.0153x vs baseline; 1.0153x over previous
import jax
import jax.numpy as jnp
from jax import lax
from jax.experimental import pallas as pl
from jax.experimental.pallas import tpu as pltpu

Z = 4
DL = 128
MAXQ = 16


def kernel(x, dest):
    t, d = x.shape
    dr = t // DL
    dest2 = dest.reshape(dr, DL)

    def body(x_ref, d_ref, out_ref, cmat_ref, cnt_s, dest_s, runn_s,
             c_send, c_recv, row_send, in_sem, own_sem, cp_sems):
        mx = lax.axis_index("x")
        my = lax.axis_index("y")
        mz = lax.axis_index("z")

        def row_rdma(src_i, dst_j, dev_z):
            return pltpu.make_async_remote_copy(
                src_ref=x_ref.at[pl.ds(src_i, 1), :],
                dst_ref=out_ref.at[pl.ds(dst_j, 1), :],
                send_sem=row_send,
                recv_sem=in_sem,
                device_id=(mx, my, dev_z),
                device_id_type=pl.DeviceIdType.MESH,
            )

        def own_copy(src_i, dst_j):
            return pltpu.make_async_copy(
                x_ref.at[pl.ds(src_i, 1), :],
                out_ref.at[pl.ds(dst_j, 1), :],
                own_sem,
            )

        barrier = pltpu.get_barrier_semaphore()
        for j in range(1, Z):
            pl.semaphore_signal(
                barrier, inc=1,
                device_id=(mx, my, lax.rem(mz + j, Z)),
                device_id_type=pl.DeviceIdType.MESH,
            )
        pl.semaphore_wait(barrier, Z - 1)

        cp_d = pltpu.make_async_copy(d_ref, dest_s, cp_sems.at[0])
        cp_d.start()

        lane = lax.broadcasted_iota(jnp.int32, (1, DL), 1)
        row = jnp.zeros((1, DL), jnp.int32)
        for r in range(Z):
            c_r = jnp.sum((d_ref[...] == r).astype(jnp.int32))
            row = row + jnp.where(lane == r, c_r, 0)
        cmat_ref[pl.ds(mz, 1), :] = row

        def cnt_rdma(dev_z):
            return pltpu.make_async_remote_copy(
                src_ref=cmat_ref.at[pl.ds(mz, 1), :],
                dst_ref=cmat_ref.at[pl.ds(mz, 1), :],
                send_sem=c_send,
                recv_sem=c_recv,
                device_id=(mx, my, dev_z),
                device_id_type=pl.DeviceIdType.MESH,
            )

        for j in range(1, Z):
            cnt_rdma(lax.rem(mz + j, Z)).start()
        for j in range(Z - 1):
            cnt_rdma(mz).wait_recv()
        for j in range(Z - 1):
            cnt_rdma(mz).wait_send()
        cp_c = pltpu.make_async_copy(cmat_ref, cnt_s, cp_sems.at[1])
        cp_c.start()
        cp_c.wait()
        cp_d.wait()

        for r in range(Z):
            base_r = jnp.int32(0)
            for s in range(Z):
                base_r = base_r + jnp.where(s < mz, cnt_s[s, r], 0)
            runn_s[r] = base_r

        def one(l, carry, rr):
            n_rem, w_rem, n_loc, w_loc = carry
            i = rr * DL + l
            dloc = dest_s[rr, l]
            pos = runn_s[dloc]
            runn_s[dloc] = pos + 1
            is_rem = (dloc != mz).astype(jnp.int32)

            @pl.when(dloc != mz)
            def _():
                row_rdma(i, pos, dloc).start()

            @pl.when(dloc == mz)
            def _():
                own_copy(i, pos).start()

            n_rem = n_rem + is_rem
            n_loc = n_loc + 1 - is_rem
            wr = (n_rem - w_rem >= MAXQ).astype(jnp.int32)
            wl = (n_loc - w_loc >= MAXQ).astype(jnp.int32)

            @pl.when(wr == 1)
            def _():
                row_rdma(0, 0, mz).wait_send()

            @pl.when(wl == 1)
            def _():
                own_copy(0, 0).wait()

            return n_rem, w_rem + wr, n_loc, w_loc + wl

        carry = (jnp.int32(0),) * 4
        for rr in range(dr):
            carry = lax.fori_loop(
                0, DL, lambda l, c, _rr=rr: one(l, c, _rr), carry
            )
        n_rem, w_rem, n_loc, w_loc = carry

        def drain(n, wait_fn):
            def step(_, c):
                wait_fn()
                return c
            lax.fori_loop(0, n, step, 0)

        drain(n_rem - w_rem, lambda: row_rdma(0, 0, mz).wait_send())
        drain(n_loc - w_loc, lambda: own_copy(0, 0).wait())

        n_in = jnp.int32(0)
        for s in range(Z):
            n_in = n_in + jnp.where(s == mz, 0, cnt_s[s, mz])
        drain(n_in, lambda: row_rdma(0, 0, mz).wait_recv())

    return pl.pallas_call(
        body,
        out_shape=jax.ShapeDtypeStruct((t, d), jnp.float32),
        in_specs=[
            pl.BlockSpec(memory_space=pltpu.VMEM),
            pl.BlockSpec(memory_space=pltpu.VMEM),
        ],
        out_specs=pl.BlockSpec(memory_space=pltpu.VMEM),
        scratch_shapes=[
            pltpu.VMEM((Z, DL), jnp.int32),
            pltpu.SMEM((Z, DL), jnp.int32),
            pltpu.SMEM((dr, DL), jnp.int32),
            pltpu.SMEM((Z,), jnp.int32),
            pltpu.SemaphoreType.DMA,
            pltpu.SemaphoreType.DMA,
            pltpu.SemaphoreType.DMA,
            pltpu.SemaphoreType.DMA,
            pltpu.SemaphoreType.DMA,
            pltpu.SemaphoreType.DMA((2,)),
        ],
        compiler_params=pltpu.CompilerParams(
            collective_id=0,
            vmem_limit_bytes=100 * 1024 * 1024,
        ),
    )(x, dest2)
